# baseline (speedup 1.0000x reference)
import functools

import jax
import jax.numpy as jnp
from jax.experimental import pallas as pl
from jax.experimental.pallas import tpu as pltpu
from jax.experimental.pallas import tpu_sc as plsc

E = 8
D = 768
V = 32000
B = 128
VT = 6400
NVT = V // VT

LANES = 16
NW = B // LANES


@functools.partial(
    pl.kernel,
    out_type=jax.ShapeDtypeStruct((B,), jnp.int32),
    mesh=plsc.VectorSubcoreMesh(core_axis_name="c", subcore_axis_name="s"),
    scratch_types=[
        pltpu.VMEM((LANES,), jnp.int32),
        pltpu.VMEM((LANES,), jnp.int32),
    ],
)
def _dispatch(ptr_hbm, idx_hbm, ptr_v, idx_v):
    wid = jax.lax.axis_index("s") * 2 + jax.lax.axis_index("c")

    @pl.when(wid < NW)
    def _():
        base = wid * LANES
        pltpu.sync_copy(ptr_hbm.at[pl.ds(base, LANES)], ptr_v)
        idx_v[...] = ptr_v[...] % E
        pltpu.sync_copy(idx_v, idx_hbm.at[pl.ds(base, LANES)])


def _router_body(idx_ref, x_ref, w_ref, b_ref, out_ref, xm_ref, oh_ref):
    v = pl.program_id(0)
    e = pl.program_id(1)

    @pl.when(v == 0)
    def _():
        idx = idx_ref[...] % E
        mask = idx == e
        xm = jnp.where(mask, x_ref[...], 0.0)
        xm_ref[e] = xm.astype(jnp.bfloat16)

        @pl.when(e == 0)
        def _():
            cols = jax.lax.broadcasted_iota(jnp.int32, (B, E), 1)
            oh_ref[...] = (idx == cols).astype(jnp.bfloat16)

    xmb = xm_ref[e]
    wb = w_ref[0].astype(jnp.bfloat16)
    acc = jax.lax.dot_general(
        xmb, wb,
        dimension_numbers=(((1,), (1,)), ((), ())),
        preferred_element_type=jnp.float32,
    )

    @pl.when(e == 0)
    def _():
        bias = jax.lax.dot_general(
            oh_ref[...], b_ref[...].astype(jnp.bfloat16),
            dimension_numbers=(((1,), (0,)), ((), ())),
            preferred_element_type=jnp.float32,
        )
        out_ref[...] = acc + bias

    @pl.when(e > 0)
    def _():
        out_ref[...] += acc


@jax.jit
def _router(idx_col, x, W, b):
    grid = (NVT, E)
    return pl.pallas_call(
        _router_body,
        grid=grid,
        in_specs=[
            pl.BlockSpec((B, 1), lambda v, e: (0, 0)),
            pl.BlockSpec((B, D), lambda v, e: (0, 0)),
            pl.BlockSpec((1, VT, D), lambda v, e: (e, v, 0)),
            pl.BlockSpec((E, VT), lambda v, e: (0, v)),
        ],
        out_specs=pl.BlockSpec((B, VT), lambda v, e: (0, v)),
        out_shape=jax.ShapeDtypeStruct((B, V), jnp.float32),
        scratch_shapes=[
            pltpu.VMEM((E, B, D), jnp.bfloat16),
            pltpu.VMEM((B, E), jnp.bfloat16),
        ],
    )(idx_col, x, W, b)


def kernel(x, pointer_addresses, W, b):
    idx = _dispatch(pointer_addresses.astype(jnp.int32))
    return _router(idx.reshape(B, 1), x, W, b)


# speedup vs baseline: 1.0036x; 1.0036x over previous
import functools

import jax
import jax.numpy as jnp
from jax.experimental import pallas as pl
from jax.experimental.pallas import tpu as pltpu
from jax.experimental.pallas import tpu_sc as plsc

E = 8
D = 768
V = 32000
B = 128
VT = 6400
NVT = V // VT

LANES = 16
NW = B // LANES


@functools.partial(
    pl.kernel,
    out_type=jax.ShapeDtypeStruct((B,), jnp.int32),
    mesh=plsc.VectorSubcoreMesh(core_axis_name="c", subcore_axis_name="s",
                                num_cores=1),
    scratch_types=[
        pltpu.VMEM((LANES,), jnp.int32),
        pltpu.VMEM((LANES,), jnp.int32),
    ],
)
def _dispatch(ptr_hbm, idx_hbm, ptr_v, idx_v):
    wid = jax.lax.axis_index("s")

    @pl.when(wid < NW)
    def _():
        base = wid * LANES
        pltpu.sync_copy(ptr_hbm.at[pl.ds(base, LANES)], ptr_v)
        idx_v[...] = ptr_v[...] % E
        pltpu.sync_copy(idx_v, idx_hbm.at[pl.ds(base, LANES)])


def _router_body(idx_ref, x_ref, w_ref, b_ref, out_ref, xm_ref, oh_ref):
    v = pl.program_id(0)
    e = pl.program_id(1)

    @pl.when(v == 0)
    def _():
        idx = idx_ref[...] % E
        mask = idx == e
        xm = jnp.where(mask, x_ref[...], 0.0)
        xm_ref[e] = xm.astype(jnp.bfloat16)

        @pl.when(e == 0)
        def _():
            cols = jax.lax.broadcasted_iota(jnp.int32, (B, E), 1)
            oh_ref[...] = (idx == cols).astype(jnp.bfloat16)

    xmb = xm_ref[e]
    wb = w_ref[0].astype(jnp.bfloat16)
    acc = jax.lax.dot_general(
        xmb, wb,
        dimension_numbers=(((1,), (1,)), ((), ())),
        preferred_element_type=jnp.float32,
    )

    @pl.when(e == 0)
    def _():
        bias = jax.lax.dot_general(
            oh_ref[...], b_ref[...].astype(jnp.bfloat16),
            dimension_numbers=(((1,), (0,)), ((), ())),
            preferred_element_type=jnp.float32,
        )
        out_ref[...] = acc + bias

    @pl.when(e > 0)
    def _():
        out_ref[...] += acc


@jax.jit
def _router(idx_col, x, W, b):
    grid = (NVT, E)
    return pl.pallas_call(
        _router_body,
        grid=grid,
        in_specs=[
            pl.BlockSpec((B, 1), lambda v, e: (0, 0)),
            pl.BlockSpec((B, D), lambda v, e: (0, 0)),
            pl.BlockSpec((1, VT, D), lambda v, e: (e, v, 0)),
            pl.BlockSpec((E, VT), lambda v, e: (0, v)),
        ],
        out_specs=pl.BlockSpec((B, VT), lambda v, e: (0, v)),
        out_shape=jax.ShapeDtypeStruct((B, V), jnp.float32),
        scratch_shapes=[
            pltpu.VMEM((E, B, D), jnp.bfloat16),
            pltpu.VMEM((B, E), jnp.bfloat16),
        ],
    )(idx_col, x, W, b)


def kernel(x, pointer_addresses, W, b):
    idx = _dispatch(pointer_addresses.astype(jnp.int32))
    return _router(idx.reshape(B, 1), x, W, b)


# speedup vs baseline: 1.0094x; 1.0057x over previous
import functools

import jax
import jax.numpy as jnp
from jax.experimental import pallas as pl
from jax.experimental.pallas import tpu as pltpu
from jax.experimental.pallas import tpu_sc as plsc

E = 8
D = 768
V = 32000
B = 128
VT = 6400
NVT = V // VT

LANES = 16
NW = B // LANES


@functools.partial(
    pl.kernel,
    out_type=jax.ShapeDtypeStruct((B,), jnp.int32),
    mesh=plsc.ScalarSubcoreMesh(axis_name="c", num_cores=1),
    scratch_types=[
        pltpu.SMEM((B,), jnp.int32),
        pltpu.SMEM((B,), jnp.int32),
    ],
)
def _dispatch(ptr_hbm, idx_hbm, ptr_s, idx_s):
    pltpu.sync_copy(ptr_hbm, ptr_s)

    def body(i, _):
        idx_s[i] = ptr_s[i] % E
        return 0

    jax.lax.fori_loop(0, B, body, 0)
    pltpu.sync_copy(idx_s, idx_hbm)


def _router_body(idx_ref, x_ref, w_ref, b_ref, out_ref, xm_ref, oh_ref):
    v = pl.program_id(0)
    e = pl.program_id(1)

    @pl.when(v == 0)
    def _():
        idx = idx_ref[...] % E
        mask = idx == e
        xm = jnp.where(mask, x_ref[...], 0.0)
        xm_ref[e] = xm.astype(jnp.bfloat16)

        @pl.when(e == 0)
        def _():
            cols = jax.lax.broadcasted_iota(jnp.int32, (B, E), 1)
            oh_ref[...] = (idx == cols).astype(jnp.bfloat16)

    xmb = xm_ref[e]
    wb = w_ref[0].astype(jnp.bfloat16)
    acc = jax.lax.dot_general(
        xmb, wb,
        dimension_numbers=(((1,), (1,)), ((), ())),
        preferred_element_type=jnp.float32,
    )

    @pl.when(e == 0)
    def _():
        bias = jax.lax.dot_general(
            oh_ref[...], b_ref[...].astype(jnp.bfloat16),
            dimension_numbers=(((1,), (0,)), ((), ())),
            preferred_element_type=jnp.float32,
        )
        out_ref[...] = acc + bias

    @pl.when(e > 0)
    def _():
        out_ref[...] += acc


@jax.jit
def _router(idx_col, x, W, b):
    grid = (NVT, E)
    return pl.pallas_call(
        _router_body,
        grid=grid,
        in_specs=[
            pl.BlockSpec((B, 1), lambda v, e: (0, 0)),
            pl.BlockSpec((B, D), lambda v, e: (0, 0)),
            pl.BlockSpec((1, VT, D), lambda v, e: (e, v, 0)),
            pl.BlockSpec((E, VT), lambda v, e: (0, v)),
        ],
        out_specs=pl.BlockSpec((B, VT), lambda v, e: (0, v)),
        out_shape=jax.ShapeDtypeStruct((B, V), jnp.float32),
        scratch_shapes=[
            pltpu.VMEM((E, B, D), jnp.bfloat16),
            pltpu.VMEM((B, E), jnp.bfloat16),
        ],
    )(idx_col, x, W, b)


def kernel(x, pointer_addresses, W, b):
    idx = _dispatch(pointer_addresses.astype(jnp.int32))
    return _router(idx.reshape(B, 1), x, W, b)


# speedup vs baseline: 1.0107x; 1.0013x over previous
import functools

import jax
import jax.numpy as jnp
from jax.experimental import pallas as pl
from jax.experimental.pallas import tpu as pltpu
from jax.experimental.pallas import tpu_sc as plsc

E = 8
D = 768
V = 32000
B = 128
VT = 6400
NVT = V // VT

LANES = 16
NW = B // LANES


@functools.partial(
    pl.kernel,
    out_type=jax.ShapeDtypeStruct((B,), jnp.int32),
    mesh=plsc.VectorSubcoreMesh(core_axis_name="c", subcore_axis_name="s",
                                num_cores=1),
    scratch_types=[
        pltpu.VMEM((LANES,), jnp.int32),
        pltpu.VMEM((LANES,), jnp.int32),
    ],
)
def _dispatch(ptr_hbm, idx_hbm, ptr_v, idx_v):
    wid = jax.lax.axis_index("s")

    @pl.when(wid < NW)
    def _():
        base = wid * LANES
        pltpu.sync_copy(ptr_hbm.at[pl.ds(base, LANES)], ptr_v)
        idx_v[...] = ptr_v[...] % E
        pltpu.sync_copy(idx_v, idx_hbm.at[pl.ds(base, LANES)])


def _router_body(idx_ref, x_ref, w_ref, b_ref, out_ref, xm_ref, oh_ref):
    v = pl.program_id(0)
    e = pl.program_id(1)

    @pl.when(v == 0)
    def _():
        row = jax.lax.broadcasted_iota(jnp.int32, (B, B), 0)
        col = jax.lax.broadcasted_iota(jnp.int32, (B, B), 1)
        eye = (row == col).astype(jnp.float32)
        idx_row = idx_ref[...].astype(jnp.float32)
        idx_col = jax.lax.dot_general(
            eye, idx_row,
            dimension_numbers=(((1,), (1,)), ((), ())),
            preferred_element_type=jnp.float32,
        )
        mask = idx_col == e.astype(jnp.float32)
        xm = jnp.where(mask, x_ref[...], 0.0)
        xm_ref[e] = xm.astype(jnp.bfloat16)

        @pl.when(e == 0)
        def _():
            cols = jax.lax.broadcasted_iota(jnp.int32, (B, E), 1)
            oh_ref[...] = (idx_col == cols.astype(jnp.float32)).astype(
                jnp.bfloat16)

    xmb = xm_ref[e]
    wb = w_ref[0].astype(jnp.bfloat16)
    acc = jax.lax.dot_general(
        xmb, wb,
        dimension_numbers=(((1,), (1,)), ((), ())),
        preferred_element_type=jnp.float32,
    )

    @pl.when(e == 0)
    def _():
        bias = jax.lax.dot_general(
            oh_ref[...], b_ref[...].astype(jnp.bfloat16),
            dimension_numbers=(((1,), (0,)), ((), ())),
            preferred_element_type=jnp.float32,
        )
        out_ref[...] = acc + bias

    @pl.when(e > 0)
    def _():
        out_ref[...] += acc


@jax.jit
def _router(idx_row, x, W, b):
    grid = (NVT, E)
    return pl.pallas_call(
        _router_body,
        grid=grid,
        in_specs=[
            pl.BlockSpec((1, B), lambda v, e: (0, 0)),
            pl.BlockSpec((B, D), lambda v, e: (0, 0)),
            pl.BlockSpec((1, VT, D), lambda v, e: (e, v, 0)),
            pl.BlockSpec((E, VT), lambda v, e: (0, v)),
        ],
        out_specs=pl.BlockSpec((B, VT), lambda v, e: (0, v)),
        out_shape=jax.ShapeDtypeStruct((B, V), jnp.float32),
        scratch_shapes=[
            pltpu.VMEM((E, B, D), jnp.bfloat16),
            pltpu.VMEM((B, E), jnp.bfloat16),
        ],
    )(idx_row, x, W, b)


def kernel(x, pointer_addresses, W, b):
    idx = _dispatch(pointer_addresses.astype(jnp.int32))
    return _router(idx.reshape(1, B), x, W, b)
